# baseline (speedup 1.0000x reference)
import functools

import jax
import jax.numpy as jnp
from jax import lax
from jax.experimental import pallas as pl
from jax.experimental.pallas import tpu as pltpu


def _leaky(x, slope=0.01):
    return jnp.where(x >= 0, x, slope * x)


def _ln(x, g, b, eps=1e-5):
    mu = jnp.mean(x, axis=-1, keepdims=True)
    var = jnp.mean((x - mu) ** 2, axis=-1, keepdims=True)
    return (x - mu) * lax.rsqrt(var + eps) * g + b


def _edge_kernel(cand_ref, col_ref, x_ref, ea_ref,
                 w1ax_ref, w1ae_ref, b1a_ref, g1_ref, be1_ref, w1b_ref, b1b_ref,
                 agg_ref, acc, cnt, g, hs, *, C, N, steps):
    step = pl.program_id(0)

    @pl.when(step == 0)
    def _init():
        acc[...] = jnp.zeros_like(acc)
        cnt[...] = jnp.zeros_like(cnt)

    def gbody(i, _):
        idx = cand_ref[0, 0, i]
        g[pl.ds(i, 1), :] = x_ref[pl.ds(idx, 1), :]
        return 0

    lax.fori_loop(0, C, gbody, 0, unroll=8)

    h = (jnp.dot(g[...], w1ax_ref[...], preferred_element_type=jnp.float32)
         + jnp.dot(ea_ref[...], w1ae_ref[...], preferred_element_type=jnp.float32)
         + b1a_ref[...])
    h = _leaky(h)
    h = _ln(h, g1_ref[...], be1_ref[...])
    hs[...] = jnp.dot(h, w1b_ref[...], preferred_element_type=jnp.float32) + b1b_ref[...]

    one8 = jnp.ones((1, 8), jnp.float32)

    def sbody(i, _):
        c = col_ref[0, 0, i]
        acc[pl.ds(c, 1), :] += hs[pl.ds(i, 1), :]
        cnt[pl.ds(c, 1), :] += one8
        return 0

    lax.fori_loop(0, C, sbody, 0)

    @pl.when(step == steps - 1)
    def _fin():
        agg_ref[...] = acc[0:N, :] / jnp.clip(cnt[0:N, 0:1], 1.0, None)


def _node_kernel(x_ref, agg_ref, b_ref, u_ref,
                 w2ax_ref, w2ag_ref, w2au_ref, b2a_ref, g2_ref, be2_ref,
                 w2b_ref, b2b_ref, out_ref, *, G):
    R = x_ref.shape[0]
    oh = (b_ref[...] == lax.broadcasted_iota(jnp.int32, (R, G), 1)).astype(jnp.float32)
    ub = jnp.dot(oh, u_ref[...], preferred_element_type=jnp.float32)
    h = (jnp.dot(x_ref[...], w2ax_ref[...], preferred_element_type=jnp.float32)
         + jnp.dot(agg_ref[...], w2ag_ref[...], preferred_element_type=jnp.float32)
         + jnp.dot(ub, w2au_ref[...], preferred_element_type=jnp.float32)
         + b2a_ref[...])
    h = _leaky(h)
    h = _ln(h, g2_ref[...], be2_ref[...])
    out_ref[...] = jnp.dot(h, w2b_ref[...], preferred_element_type=jnp.float32) + b2b_ref[...]


def kernel(x, edge_index, edge_attr, u, batch,
           W1a, b1a, ln1_g, ln1_b, W1b, b1b,
           W2a, b2a, ln2_g, ln2_b, W2b, b2b):
    N, DN = x.shape
    E, DE = edge_attr.shape
    G, DG = u.shape
    DO = W1a.shape[0]

    C = 3200 if E >= 3200 else ((E + 7) // 8) * 8
    Ep = ((E + C - 1) // C) * C
    steps = Ep // C
    cand = edge_index[0]
    col = edge_index[1]
    if Ep != E:
        cand = jnp.concatenate([cand, jnp.zeros((Ep - E,), jnp.int32)])
        col = jnp.concatenate([col, jnp.full((Ep - E,), N, jnp.int32)])
        edge_attr = jnp.concatenate(
            [edge_attr, jnp.zeros((Ep - E, DE), jnp.float32)], axis=0)
    cand3 = cand.reshape(steps, 1, C)
    col3 = col.reshape(steps, 1, C)

    w1ax = W1a[:, :DN].T
    w1ae = W1a[:, DN:].T

    full = lambda shape: pl.BlockSpec(shape, lambda i: (0,) * len(shape))
    agg = pl.pallas_call(
        functools.partial(_edge_kernel, C=C, N=N, steps=steps),
        grid=(steps,),
        in_specs=[
            pl.BlockSpec((1, 1, C), lambda i: (i, 0, 0), memory_space=pltpu.SMEM),
            pl.BlockSpec((1, 1, C), lambda i: (i, 0, 0), memory_space=pltpu.SMEM),
            full((N, DN)),
            pl.BlockSpec((C, DE), lambda i: (i, 0)),
            full((DN, DO)),
            full((DE, DO)),
            full((1, DO)),
            full((1, DO)),
            full((1, DO)),
            full((DO, DO)),
            full((1, DO)),
        ],
        out_specs=full((N, DO)),
        out_shape=jax.ShapeDtypeStruct((N, DO), jnp.float32),
        scratch_shapes=[
            pltpu.VMEM((N + 8, DO), jnp.float32),
            pltpu.VMEM((N + 8, 8), jnp.float32),
            pltpu.VMEM((C, DN), jnp.float32),
            pltpu.VMEM((C, DO), jnp.float32),
        ],
    )(cand3, col3, x, edge_attr, w1ax, w1ae,
      b1a[None, :], ln1_g[None, :], ln1_b[None, :], W1b.T, b1b[None, :])

    R = 1000
    if N % R != 0:
        R = 8
    nsteps = N // R
    out = pl.pallas_call(
        functools.partial(_node_kernel, G=G),
        grid=(nsteps,),
        in_specs=[
            pl.BlockSpec((R, DN), lambda i: (i, 0)),
            pl.BlockSpec((R, DO), lambda i: (i, 0)),
            pl.BlockSpec((R, 1), lambda i: (i, 0)),
            full((G, DG)),
            full((DN, DO)),
            full((DO, DO)),
            full((DG, DO)),
            full((1, DO)),
            full((1, DO)),
            full((1, DO)),
            full((DO, DO)),
            full((1, DO)),
        ],
        out_specs=pl.BlockSpec((R, DO), lambda i: (i, 0)),
        out_shape=jax.ShapeDtypeStruct((N, DO), jnp.float32),
    )(x, agg, batch[:, None], u,
      W2a[:, :DN].T, W2a[:, DN:DN + DO].T, W2a[:, DN + DO:].T,
      b2a[None, :], ln2_g[None, :], ln2_b[None, :], W2b.T, b2b[None, :])
    return out


# speedup vs baseline: 1.3628x; 1.3628x over previous
import functools

import jax
import jax.numpy as jnp
from jax import lax
from jax.experimental import pallas as pl
from jax.experimental.pallas import tpu as pltpu
from jax.experimental.pallas import tpu_sc as plsc

NC, NS = 2, 16
NW = NC * NS
LG = 128
KG = 4
KS = 1


def _leaky(x, slope=0.01):
    return jnp.where(x >= 0, x, slope * x)


def _ln(x, g, b, eps=1e-5):
    mu = jnp.mean(x, axis=-1, keepdims=True)
    var = jnp.mean((x - mu) ** 2, axis=-1, keepdims=True)
    return (x - mu) * lax.rsqrt(var + eps) * g + b


def _sc_gather_body(x_hbm, cand_hbm, gx_hbm, idx_v, rows_v, sem, *, groups):
    c = lax.axis_index("c")
    s = lax.axis_index("s")
    wid = s * NC + c
    row0 = wid * groups * KG

    def step(g, carry):
        r = row0 + g * KG
        pltpu.sync_copy(cand_hbm.at[pl.ds(r, KG), :], idx_v)
        descs = []
        for j in range(KG):
            descs.append(pltpu.async_copy(
                x_hbm.at[idx_v.at[j]], rows_v.at[pl.ds(j * LG, LG), :], sem))
        for d in descs:
            d.wait()
        pltpu.sync_copy(rows_v, gx_hbm.at[pl.ds(r * LG, KG * LG), :])
        return carry

    lax.fori_loop(0, groups, step, 0)


def _sc_gather(x, cand2d, groups, DN):
    Ep = cand2d.shape[0] * LG
    fn = pl.kernel(
        functools.partial(_sc_gather_body, groups=groups),
        out_type=jax.ShapeDtypeStruct((Ep, DN), jnp.float32),
        mesh=plsc.VectorSubcoreMesh(core_axis_name="c", subcore_axis_name="s"),
        scratch_types=[
            pltpu.VMEM((KG, LG), jnp.int32),
            pltpu.VMEM((KG * LG, DN), jnp.float32),
            pltpu.SemaphoreType.DMA,
        ],
    )
    return fn(x, cand2d)


def _sc_scatter_body(h01_hbm, col_hbm, zrow_hbm, agg_hbm, idx_v, rows_v, acc,
                     *, groups, rows_per_sub, HW):
    c = lax.axis_index("c")
    s = lax.axis_index("s")

    pltpu.sync_copy(zrow_hbm, acc.at[pl.ds(s * rows_per_sub, rows_per_sub), :])
    plsc.subcore_barrier()

    def step(g, carry):
        r = (s * groups + g) * KS
        pltpu.sync_copy(col_hbm.at[pl.ds(r, KS), :], idx_v)
        pltpu.sync_copy(h01_hbm.at[c, pl.ds(r * LG, KS * LG), :], rows_v)
        for j in range(KS):
            pltpu.sync_copy(rows_v.at[pl.ds(j * LG, LG), :],
                            acc.at[idx_v.at[j]], add=True)
        return carry

    lax.fori_loop(0, groups, step, 0)
    plsc.subcore_barrier()
    pltpu.sync_copy(acc.at[pl.ds(s * rows_per_sub, rows_per_sub), :],
                    agg_hbm.at[c, pl.ds(s * rows_per_sub, rows_per_sub), :])


def _sc_scatter(h01, col2d, Np, HW):
    rows_per_sub = Np // NS
    groups = col2d.shape[0] // (NS * KS)
    zrow = jnp.zeros((rows_per_sub, HW), jnp.float32)
    fn = pl.kernel(
        functools.partial(_sc_scatter_body, groups=groups,
                          rows_per_sub=rows_per_sub, HW=HW),
        out_type=jax.ShapeDtypeStruct((NC, Np, HW), jnp.float32),
        mesh=plsc.VectorSubcoreMesh(core_axis_name="c", subcore_axis_name="s"),
        scratch_types=[
            pltpu.VMEM((KS, LG), jnp.int32),
            pltpu.VMEM((KS * LG, HW), jnp.float32),
            pltpu.VMEM_SHARED((Np, HW), jnp.float32),
        ],
        compiler_params=pltpu.CompilerParams(use_tc_tiling_on_sc=False),
    )
    return fn(h01, col2d, zrow)


def _edge_kernel(gx_ref, ea_ref, w1ax_ref, w1ae_ref, b1a_ref, g1_ref, be1_ref,
                 w1b_ref, b1b_ref, h01_ref, *, HW):
    C = gx_ref.shape[0]
    h = (jnp.dot(gx_ref[...], w1ax_ref[...], preferred_element_type=jnp.float32)
         + jnp.dot(ea_ref[...], w1ae_ref[...], preferred_element_type=jnp.float32)
         + b1a_ref[...])
    h = _leaky(h)
    h = _ln(h, g1_ref[...], be1_ref[...])
    h2 = jnp.dot(h, w1b_ref[...], preferred_element_type=jnp.float32) + b1b_ref[...]
    DO = h2.shape[1]
    h01_ref[0, :, :] = h2[:, :HW]
    h01_ref[1, :, :] = jnp.concatenate(
        [h2[:, HW:], jnp.ones((C, 2 * HW - DO), jnp.float32)], axis=1)


def _node_kernel(x_ref, agg_ref, b_ref, u_ref,
                 w2ax_ref, w2ag_ref, w2au_ref, b2a_ref, g2_ref, be2_ref,
                 w2b_ref, b2b_ref, out_ref, *, G, HW, DO):
    R = x_ref.shape[0]
    a0 = agg_ref[0, :, :]
    a1 = agg_ref[1, :, :]
    cnt = a1[:, DO - HW:DO - HW + 1]
    inv = 1.0 / jnp.clip(cnt, 1.0, None)
    aggm = jnp.concatenate([a0, a1[:, :DO - HW]], axis=1) * inv
    oh = (b_ref[...] == lax.broadcasted_iota(jnp.int32, (R, G), 1)).astype(jnp.float32)
    ub = jnp.dot(oh, u_ref[...], preferred_element_type=jnp.float32)
    h = (jnp.dot(x_ref[...], w2ax_ref[...], preferred_element_type=jnp.float32)
         + jnp.dot(aggm, w2ag_ref[...], preferred_element_type=jnp.float32)
         + jnp.dot(ub, w2au_ref[...], preferred_element_type=jnp.float32)
         + b2a_ref[...])
    h = _leaky(h)
    h = _ln(h, g2_ref[...], be2_ref[...])
    out_ref[...] = jnp.dot(h, w2b_ref[...], preferred_element_type=jnp.float32) + b2b_ref[...]


def kernel(x, edge_index, edge_attr, u, batch,
           W1a, b1a, ln1_g, ln1_b, W1b, b1b,
           W2a, b2a, ln2_g, ln2_b, W2b, b2b):
    N, DN = x.shape
    E, DE = edge_attr.shape
    G, DG = u.shape
    DO = W1a.shape[0]
    HW = 144
    Np = ((N + 8 + NS * 8 - 1) // (NS * 8)) * (NS * 8)

    CH = NW * KG * LG
    Ep = ((E + CH - 1) // CH) * CH
    cand = edge_index[0]
    col = edge_index[1]
    if Ep != E:
        pad = Ep - E
        cand = jnp.concatenate([cand, jnp.zeros((pad,), jnp.int32)])
        trash = N + (jnp.arange(pad, dtype=jnp.int32) % 8)
        col = jnp.concatenate([col, trash])
        edge_attr = jnp.concatenate(
            [edge_attr, jnp.zeros((pad, DE), jnp.float32)], axis=0)
    cand2d = cand.reshape(Ep // LG, LG)
    col2d = col.reshape(Ep // LG, LG)
    groups = Ep // (NW * KG * LG)

    gx = _sc_gather(x, cand2d, groups, DN)

    C = 4096
    steps = Ep // C
    full = lambda shape: pl.BlockSpec(shape, lambda i: (0,) * len(shape))
    h01 = pl.pallas_call(
        functools.partial(_edge_kernel, HW=HW),
        grid=(steps,),
        in_specs=[
            pl.BlockSpec((C, DN), lambda i: (i, 0)),
            pl.BlockSpec((C, DE), lambda i: (i, 0)),
            full((DN, DO)),
            full((DE, DO)),
            full((1, DO)),
            full((1, DO)),
            full((1, DO)),
            full((DO, DO)),
            full((1, DO)),
        ],
        out_specs=pl.BlockSpec((NC, C, HW), lambda i: (0, i, 0)),
        out_shape=jax.ShapeDtypeStruct((NC, Ep, HW), jnp.float32),
    )(gx, edge_attr, W1a[:, :DN].T, W1a[:, DN:].T,
      b1a[None, :], ln1_g[None, :], ln1_b[None, :], W1b.T, b1b[None, :])

    agg01 = _sc_scatter(h01, col2d, Np, HW)

    R = 1000
    if N % R != 0:
        R = 8
    nsteps = N // R
    out = pl.pallas_call(
        functools.partial(_node_kernel, G=G, HW=HW, DO=DO),
        grid=(nsteps,),
        in_specs=[
            pl.BlockSpec((R, DN), lambda i: (i, 0)),
            pl.BlockSpec((NC, R, HW), lambda i: (0, i, 0)),
            pl.BlockSpec((R, 1), lambda i: (i, 0)),
            full((G, DG)),
            full((DN, DO)),
            full((DO, DO)),
            full((DG, DO)),
            full((1, DO)),
            full((1, DO)),
            full((1, DO)),
            full((DO, DO)),
            full((1, DO)),
        ],
        out_specs=pl.BlockSpec((R, DO), lambda i: (i, 0)),
        out_shape=jax.ShapeDtypeStruct((N, DO), jnp.float32),
    )(x, agg01, batch[:, None], u,
      W2a[:, :DN].T, W2a[:, DN:DN + DO].T, W2a[:, DN + DO:].T,
      b2a[None, :], ln2_g[None, :], ln2_b[None, :], W2b.T, b2b[None, :])
    return out


# speedup vs baseline: 1.4668x; 1.0763x over previous
import functools

import jax
import jax.numpy as jnp
from jax import lax
from jax.experimental import pallas as pl
from jax.experimental.pallas import tpu as pltpu
from jax.experimental.pallas import tpu_sc as plsc

NC, NS = 2, 16
NW = NC * NS
LG = 128
KG = 4
KS = 1


def _leaky(x, slope=0.01):
    return jnp.where(x >= 0, x, slope * x)


def _ln(x, g, b, eps=1e-5):
    mu = jnp.mean(x, axis=-1, keepdims=True)
    var = jnp.mean((x - mu) ** 2, axis=-1, keepdims=True)
    return (x - mu) * lax.rsqrt(var + eps) * g + b


def _sc_gather_body(x_hbm, cand_hbm, gx_hbm, idx_v, rows_v, sem, *, groups):
    c = lax.axis_index("c")
    s = lax.axis_index("s")
    wid = s * NC + c
    row0 = wid * groups * KG

    def step(g, carry):
        r = row0 + g * KG
        pltpu.sync_copy(cand_hbm.at[pl.ds(r, KG), :], idx_v)
        descs = []
        for j in range(KG):
            descs.append(pltpu.async_copy(
                x_hbm.at[idx_v.at[j]], rows_v.at[pl.ds(j * LG, LG), :], sem))
        for d in descs:
            d.wait()
        pltpu.sync_copy(rows_v, gx_hbm.at[pl.ds(r * LG, KG * LG), :])
        return carry

    lax.fori_loop(0, groups, step, 0)


def _sc_gather(x, cand2d, groups, DN):
    Ep = cand2d.shape[0] * LG
    fn = pl.kernel(
        functools.partial(_sc_gather_body, groups=groups),
        out_type=jax.ShapeDtypeStruct((Ep, DN), jnp.float32),
        mesh=plsc.VectorSubcoreMesh(core_axis_name="c", subcore_axis_name="s"),
        scratch_types=[
            pltpu.VMEM((KG, LG), jnp.int32),
            pltpu.VMEM((KG * LG, DN), jnp.float32),
            pltpu.SemaphoreType.DMA,
        ],
    )
    return fn(x, cand2d)


def _sc_scatter_body(h01_hbm, col_hbm, zrow_hbm, agg_hbm, idx_v, rows_v, sem,
                     acc, *, groups, rows_per_sub, HW):
    c = lax.axis_index("c")
    s = lax.axis_index("s")

    pltpu.sync_copy(zrow_hbm, acc.at[pl.ds(s * rows_per_sub, rows_per_sub), :])
    plsc.subcore_barrier()

    def fire(g, b):
        r = s * groups + g
        pltpu.async_copy(col_hbm.at[r, :], idx_v.at[b], sem.at[b])
        pltpu.async_copy(h01_hbm.at[c, pl.ds(r * LG, LG), :], rows_v.at[b],
                         sem.at[b])

    def drain(g, b):
        r = s * groups + g
        pltpu.make_async_copy(col_hbm.at[r, :], idx_v.at[b], sem.at[b]).wait()
        pltpu.make_async_copy(h01_hbm.at[c, pl.ds(r * LG, LG), :], rows_v.at[b],
                              sem.at[b]).wait()

    fire(0, 0)

    def step(g, carry):
        b = lax.rem(g, 2)
        drain(g, b)

        @pl.when(g + 1 < groups)
        def _():
            fire(g + 1, 1 - b)

        pltpu.sync_copy(rows_v.at[b], acc.at[idx_v.at[b]], add=True)
        return carry

    lax.fori_loop(0, groups, step, 0)
    plsc.subcore_barrier()
    pltpu.sync_copy(acc.at[pl.ds(s * rows_per_sub, rows_per_sub), :],
                    agg_hbm.at[c, pl.ds(s * rows_per_sub, rows_per_sub), :])


def _sc_scatter(h01, col2d, Np, HW):
    rows_per_sub = Np // NS
    groups = col2d.shape[0] // NS
    zrow = jnp.zeros((rows_per_sub, HW), jnp.float32)
    fn = pl.kernel(
        functools.partial(_sc_scatter_body, groups=groups,
                          rows_per_sub=rows_per_sub, HW=HW),
        out_type=jax.ShapeDtypeStruct((NC, Np, HW), jnp.float32),
        mesh=plsc.VectorSubcoreMesh(core_axis_name="c", subcore_axis_name="s"),
        scratch_types=[
            pltpu.VMEM((2, LG), jnp.int32),
            pltpu.VMEM((2, LG, HW), jnp.float32),
            pltpu.SemaphoreType.DMA((2,)),
            pltpu.VMEM_SHARED((Np, HW), jnp.float32),
        ],
        compiler_params=pltpu.CompilerParams(use_tc_tiling_on_sc=False),
    )
    return fn(h01, col2d, zrow)


def _edge_kernel(gx_ref, ea_ref, w1ax_ref, w1ae_ref, b1a_ref, g1_ref, be1_ref,
                 w1b_ref, b1b_ref, h01_ref, *, HW):
    C = gx_ref.shape[0]
    h = (jnp.dot(gx_ref[...], w1ax_ref[...], preferred_element_type=jnp.float32)
         + jnp.dot(ea_ref[...], w1ae_ref[...], preferred_element_type=jnp.float32)
         + b1a_ref[...])
    h = _leaky(h)
    h = _ln(h, g1_ref[...], be1_ref[...])
    h2 = jnp.dot(h, w1b_ref[...], preferred_element_type=jnp.float32) + b1b_ref[...]
    DO = h2.shape[1]
    h01_ref[0, :, :] = h2[:, :HW]
    h01_ref[1, :, :] = jnp.concatenate(
        [h2[:, HW:], jnp.ones((C, 2 * HW - DO), jnp.float32)], axis=1)


def _node_kernel(x_ref, agg_ref, b_ref, u_ref,
                 w2ax_ref, w2ag_ref, w2au_ref, b2a_ref, g2_ref, be2_ref,
                 w2b_ref, b2b_ref, out_ref, *, G, HW, DO):
    R = x_ref.shape[0]
    a0 = agg_ref[0, :, :]
    a1 = agg_ref[1, :, :]
    cnt = a1[:, DO - HW:DO - HW + 1]
    inv = 1.0 / jnp.clip(cnt, 1.0, None)
    aggm = jnp.concatenate([a0, a1[:, :DO - HW]], axis=1) * inv
    oh = (b_ref[...] == lax.broadcasted_iota(jnp.int32, (R, G), 1)).astype(jnp.float32)
    ub = jnp.dot(oh, u_ref[...], preferred_element_type=jnp.float32)
    h = (jnp.dot(x_ref[...], w2ax_ref[...], preferred_element_type=jnp.float32)
         + jnp.dot(aggm, w2ag_ref[...], preferred_element_type=jnp.float32)
         + jnp.dot(ub, w2au_ref[...], preferred_element_type=jnp.float32)
         + b2a_ref[...])
    h = _leaky(h)
    h = _ln(h, g2_ref[...], be2_ref[...])
    out_ref[...] = jnp.dot(h, w2b_ref[...], preferred_element_type=jnp.float32) + b2b_ref[...]


def kernel(x, edge_index, edge_attr, u, batch,
           W1a, b1a, ln1_g, ln1_b, W1b, b1b,
           W2a, b2a, ln2_g, ln2_b, W2b, b2b):
    N, DN = x.shape
    E, DE = edge_attr.shape
    G, DG = u.shape
    DO = W1a.shape[0]
    HW = 144
    Np = ((N + 8 + NS * 8 - 1) // (NS * 8)) * (NS * 8)

    CH = NW * KG * LG
    Ep = ((E + CH - 1) // CH) * CH
    cand = edge_index[0]
    col = edge_index[1]
    if Ep != E:
        pad = Ep - E
        cand = jnp.concatenate([cand, jnp.zeros((pad,), jnp.int32)])
        trash = N + (jnp.arange(pad, dtype=jnp.int32) % 8)
        col = jnp.concatenate([col, trash])
        edge_attr = jnp.concatenate(
            [edge_attr, jnp.zeros((pad, DE), jnp.float32)], axis=0)
    cand2d = cand.reshape(Ep // LG, LG)
    col2d = col.reshape(Ep // LG, LG)
    groups = Ep // (NW * KG * LG)

    gx = _sc_gather(x, cand2d, groups, DN)

    C = 4096
    steps = Ep // C
    full = lambda shape: pl.BlockSpec(shape, lambda i: (0,) * len(shape))
    h01 = pl.pallas_call(
        functools.partial(_edge_kernel, HW=HW),
        grid=(steps,),
        in_specs=[
            pl.BlockSpec((C, DN), lambda i: (i, 0)),
            pl.BlockSpec((C, DE), lambda i: (i, 0)),
            full((DN, DO)),
            full((DE, DO)),
            full((1, DO)),
            full((1, DO)),
            full((1, DO)),
            full((DO, DO)),
            full((1, DO)),
        ],
        out_specs=pl.BlockSpec((NC, C, HW), lambda i: (0, i, 0)),
        out_shape=jax.ShapeDtypeStruct((NC, Ep, HW), jnp.float32),
    )(gx, edge_attr, W1a[:, :DN].T, W1a[:, DN:].T,
      b1a[None, :], ln1_g[None, :], ln1_b[None, :], W1b.T, b1b[None, :])

    agg01 = _sc_scatter(h01, col2d, Np, HW)

    R = 1000
    if N % R != 0:
        R = 8
    nsteps = N // R
    out = pl.pallas_call(
        functools.partial(_node_kernel, G=G, HW=HW, DO=DO),
        grid=(nsteps,),
        in_specs=[
            pl.BlockSpec((R, DN), lambda i: (i, 0)),
            pl.BlockSpec((NC, R, HW), lambda i: (0, i, 0)),
            pl.BlockSpec((R, 1), lambda i: (i, 0)),
            full((G, DG)),
            full((DN, DO)),
            full((DO, DO)),
            full((DG, DO)),
            full((1, DO)),
            full((1, DO)),
            full((1, DO)),
            full((DO, DO)),
            full((1, DO)),
        ],
        out_specs=pl.BlockSpec((R, DO), lambda i: (i, 0)),
        out_shape=jax.ShapeDtypeStruct((N, DO), jnp.float32),
    )(x, agg01, batch[:, None], u,
      W2a[:, :DN].T, W2a[:, DN:DN + DO].T, W2a[:, DN + DO:].T,
      b2a[None, :], ln2_g[None, :], ln2_b[None, :], W2b.T, b2b[None, :])
    return out
